# baseline (speedup 1.0000x reference)
import functools

import jax
import jax.numpy as jnp
from jax import lax
from jax.experimental import pallas as pl
from jax.experimental.pallas import tpu as pltpu
from jax.experimental.pallas import tpu_sc as plsc

VOCAB = 1000000
EMBED = 32
B_TOTAL = 4096 * 200
NUM_CORES = 2
NUM_SUBCORES = 16
NW = NUM_CORES * NUM_SUBCORES
B_PER_W = B_TOTAL // NW
CHUNK = 1600
N_CHUNKS = B_PER_W // CHUNK

BV = 512
NBLK = 1954
T4_ROWS = NBLK * 128

_mesh = plsc.VectorSubcoreMesh(core_axis_name="c", subcore_axis_name="s")


def _relayout_body(tt_ref, out_ref):
    t = tt_ref[...].T
    out_ref[...] = jnp.concatenate(
        [t[128 * c:128 * (c + 1), :] for c in range(4)], axis=1)


_relayout_table = pl.pallas_call(
    _relayout_body,
    grid=(NBLK,),
    in_specs=[pl.BlockSpec((32, BV), lambda i: (0, i))],
    out_specs=pl.BlockSpec((128, 128), lambda i: (i, 0)),
    out_shape=jax.ShapeDtypeStruct((T4_ROWS, 128), jnp.float32),
)


@functools.partial(
    pl.kernel,
    mesh=_mesh,
    out_type=jax.ShapeDtypeStruct((B_TOTAL, EMBED), jnp.float32),
    scratch_types=[
        pltpu.VMEM((B_PER_W,), jnp.int32),
        pltpu.VMEM((CHUNK, EMBED), jnp.float32),
        pltpu.VMEM((CHUNK, EMBED), jnp.float32),
        pltpu.SemaphoreType.DMA,
        pltpu.SemaphoreType.DMA,
        pltpu.SemaphoreType.DMA,
        pltpu.SemaphoreType.DMA,
    ],
    compiler_params=pltpu.CompilerParams(use_tc_tiling_on_sc=False),
)
def _embed_lookup(idx_hbm, table_hbm, out_hbm,
                  idx_v, rows0, rows1, sg0, sg1, so0, so1):
    wid = lax.axis_index("s") * NUM_CORES + lax.axis_index("c")
    base = wid * B_PER_W

    rows_v = (rows0, rows1)
    sem_g = (sg0, sg1)
    sem_o = (so0, so1)

    def transform(j):
        def body(i, carry):
            off = j * CHUNK + i * 16
            v = idx_v[pl.ds(off, 16)]
            r = (v & -512) + ((v & 127) << 2) + ((v >> 7) & 3)
            idx_v[pl.ds(off, 16)] = r
            return carry
        lax.fori_loop(0, CHUNK // 16, body, 0)

    def gather(j, b):
        return pltpu.make_async_copy(
            table_hbm.at[idx_v.at[pl.ds(j * CHUNK, CHUNK)]], rows_v[b], sem_g[b])

    def store(j, b):
        return pltpu.make_async_copy(
            rows_v[b], out_hbm.at[pl.ds(base + j * CHUNK, CHUNK)], sem_o[b])

    pltpu.sync_copy(idx_hbm.at[pl.ds(base, B_PER_W)], idx_v)

    transform(0)
    gather(0, 0).start()
    for j in range(N_CHUNKS):
        b = j & 1
        nb = b ^ 1
        if j + 1 < N_CHUNKS:
            if j >= 1:
                store(j - 1, nb).wait()
            transform(j + 1)
            gather(j + 1, nb).start()
        gather(j, b).wait()
        store(j, b).start()
    store(N_CHUNKS - 2, 0).wait()
    store(N_CHUNKS - 1, 1).wait()


def kernel(inputs, table):
    t4 = _relayout_table(table.T)
    tview = t4.reshape(T4_ROWS * 4, EMBED)
    flat_idx = inputs.reshape(-1)
    out = _embed_lookup(flat_idx, tview)
    return out.reshape(inputs.shape + (EMBED,))


# speedup vs baseline: 2.1863x; 2.1863x over previous
import functools

import jax
import jax.numpy as jnp
from jax import lax
from jax.experimental import pallas as pl
from jax.experimental.pallas import tpu as pltpu
from jax.experimental.pallas import tpu_sc as plsc

VOCAB = 1000000
EMBED = 32
B_TOTAL = 4096 * 200
NUM_CORES = 2
NUM_SUBCORES = 16
NW = NUM_CORES * NUM_SUBCORES
B_PER_W = B_TOTAL // NW
CHUNK = 1600
N_CHUNKS = B_PER_W // CHUNK

BV = 8192
SUB = BV // 512
NBLK = 123
T4_ROWS = NBLK * BV // 4

_mesh = plsc.VectorSubcoreMesh(core_axis_name="c", subcore_axis_name="s")


def _relayout_body(tt_ref, out_ref):
    t = tt_ref[...].T
    out_ref[...] = jnp.concatenate(
        [jnp.concatenate(
            [t[512 * q + 128 * c:512 * q + 128 * (c + 1), :]
             for c in range(4)], axis=1)
         for q in range(SUB)], axis=0)


_relayout_table = pl.pallas_call(
    _relayout_body,
    grid=(NBLK,),
    in_specs=[pl.BlockSpec((32, BV), lambda i: (0, i))],
    out_specs=pl.BlockSpec((BV // 4, 128), lambda i: (i, 0)),
    out_shape=jax.ShapeDtypeStruct((T4_ROWS, 128), jnp.float32),
)


@functools.partial(
    pl.kernel,
    mesh=_mesh,
    out_type=jax.ShapeDtypeStruct((B_TOTAL, EMBED), jnp.float32),
    scratch_types=[
        pltpu.VMEM((B_PER_W,), jnp.int32),
        pltpu.VMEM((CHUNK, EMBED), jnp.float32),
        pltpu.VMEM((CHUNK, EMBED), jnp.float32),
        pltpu.SemaphoreType.DMA,
        pltpu.SemaphoreType.DMA,
        pltpu.SemaphoreType.DMA,
        pltpu.SemaphoreType.DMA,
    ],
    compiler_params=pltpu.CompilerParams(use_tc_tiling_on_sc=False),
)
def _embed_lookup(idx_hbm, table_hbm, out_hbm,
                  idx_v, rows0, rows1, sg0, sg1, so0, so1):
    wid = lax.axis_index("s") * NUM_CORES + lax.axis_index("c")
    base = wid * B_PER_W

    rows_v = (rows0, rows1)
    sem_g = (sg0, sg1)
    sem_o = (so0, so1)

    def transform(j):
        def body(i, carry):
            off = j * CHUNK + i * 16
            v = idx_v[pl.ds(off, 16)]
            r = (v & -512) + ((v & 127) << 2) + ((v >> 7) & 3)
            idx_v[pl.ds(off, 16)] = r
            return carry
        lax.fori_loop(0, CHUNK // 16, body, 0)

    def gather(j, b):
        return pltpu.make_async_copy(
            table_hbm.at[idx_v.at[pl.ds(j * CHUNK, CHUNK)]], rows_v[b], sem_g[b])

    def store(j, b):
        return pltpu.make_async_copy(
            rows_v[b], out_hbm.at[pl.ds(base + j * CHUNK, CHUNK)], sem_o[b])

    pltpu.sync_copy(idx_hbm.at[pl.ds(base, B_PER_W)], idx_v)

    transform(0)
    gather(0, 0).start()
    for j in range(N_CHUNKS):
        b = j & 1
        nb = b ^ 1
        if j + 1 < N_CHUNKS:
            if j >= 1:
                store(j - 1, nb).wait()
            transform(j + 1)
            gather(j + 1, nb).start()
        gather(j, b).wait()
        store(j, b).start()
    store(N_CHUNKS - 2, 0).wait()
    store(N_CHUNKS - 1, 1).wait()


def kernel(inputs, table):
    t4 = _relayout_table(table.T)
    tview = t4.reshape(T4_ROWS * 4, EMBED)
    flat_idx = inputs.reshape(-1)
    out = _embed_lookup(flat_idx, tview)
    return out.reshape(inputs.shape + (EMBED,))
